# Initial kernel scaffold; baseline (speedup 1.0000x reference)
import functools

import jax
import jax.numpy as jnp
from jax import lax
from jax.experimental import pallas as pl
from jax.experimental.pallas import tpu as pltpu
from jax.experimental.pallas import tpu_sc as plsc


def _sc_kernel(E, D, R, CHUNK):
  info = plsc.get_sparse_core_info()
  NC, NS, L = info.num_cores, info.num_subcores, info.num_lanes
  NW = NC * NS
  assert E % NW == 0
  e_per_w = E // NW
  assert e_per_w % CHUNK == 0 and CHUNK % L == 0
  n_chunks = e_per_w // CHUNK
  n_groups = CHUNK // L

  mesh = plsc.VectorSubcoreMesh(core_axis_name="c", subcore_axis_name="s")

  @functools.partial(
      pl.kernel,
      mesh=mesh,
      out_type=jax.ShapeDtypeStruct((E,), jnp.float32),
      scratch_types=[
          pltpu.VMEM((e_per_w,), jnp.int32),
          pltpu.VMEM((e_per_w,), jnp.int32),
          pltpu.VMEM((e_per_w,), jnp.int32),
          pltpu.VMEM((R, D), jnp.float32),
          pltpu.VMEM((CHUNK, D), jnp.float32),
          pltpu.VMEM((CHUNK, D), jnp.float32),
          pltpu.VMEM((e_per_w,), jnp.float32),
          pltpu.SemaphoreType.DMA,
      ],
  )
  def k(x_hbm, ei_hbm, et_hbm, w_hbm, out_hbm,
        sidx, didx, et_v, w_v, srows, drows, out_v, sem):
    wid = lax.axis_index("s") * NC + lax.axis_index("c")
    base = wid * e_per_w

    pltpu.sync_copy(ei_hbm.at[0, pl.ds(base, e_per_w)], sidx)
    pltpu.sync_copy(ei_hbm.at[1, pl.ds(base, e_per_w)], didx)
    pltpu.sync_copy(et_hbm.at[pl.ds(base, e_per_w)], et_v)
    pltpu.sync_copy(w_hbm, w_v)

    lane = lax.broadcasted_iota(jnp.int32, (L,), 0)

    def chunk_body(ch, carry):
      cbase = ch * CHUNK
      cp1 = pltpu.async_copy(x_hbm.at[sidx.at[pl.ds(cbase, CHUNK)]], srows, sem)
      cp2 = pltpu.async_copy(x_hbm.at[didx.at[pl.ds(cbase, CHUNK)]], drows, sem)
      cp1.wait()
      cp2.wait()

      for g in range(n_groups):
        eoff = cbase + g * L
        et16 = et_v[pl.ds(eoff, L)]
        erow = lane + g * L

        def d_body(d, acc):
          dv = jnp.full((L,), d, dtype=jnp.int32)
          s = plsc.load_gather(srows, [erow, dv])
          t = plsc.load_gather(drows, [erow, dv])
          w = plsc.load_gather(w_v, [et16, dv])
          return acc + s * t * w

        acc = lax.fori_loop(0, D, d_body, jnp.zeros((L,), jnp.float32))
        out_v[pl.ds(eoff, L)] = 1.0 / (1.0 + jnp.exp(-acc))
      return carry

    lax.fori_loop(0, n_chunks, chunk_body, 0)
    pltpu.sync_copy(out_v, out_hbm.at[pl.ds(base, e_per_w)])

  return k


def kernel(x, edge_index, edge_type, weight):
  E = edge_type.shape[0]
  D = x.shape[1]
  R = weight.shape[0]
  k = _sc_kernel(E, D, R, CHUNK=80)
  return k(x, edge_index, edge_type, weight)


# speedup vs baseline: 1.0990x; 1.0990x over previous
import functools

import jax
import jax.numpy as jnp
from jax import lax
from jax.experimental import pallas as pl
from jax.experimental.pallas import tpu as pltpu
from jax.experimental.pallas import tpu_sc as plsc


def _sc_kernel(E, D, R, CHUNK):
  info = plsc.get_sparse_core_info()
  NC, NS, L = info.num_cores, info.num_subcores, info.num_lanes
  NW = NC * NS
  assert E % NW == 0
  e_per_w = E // NW
  assert e_per_w % CHUNK == 0 and CHUNK % L == 0
  n_chunks = e_per_w // CHUNK
  n_groups = CHUNK // L

  mesh = plsc.VectorSubcoreMesh(core_axis_name="c", subcore_axis_name="s")

  @functools.partial(
      pl.kernel,
      mesh=mesh,
      compiler_params=pltpu.CompilerParams(needs_layout_passes=False),
      out_type=jax.ShapeDtypeStruct((E,), jnp.float32),
      scratch_types=[
          pltpu.VMEM((e_per_w,), jnp.int32),
          pltpu.VMEM((e_per_w,), jnp.int32),
          pltpu.VMEM((e_per_w,), jnp.int32),
          pltpu.VMEM((R, D), jnp.float32),
          pltpu.VMEM((CHUNK, D), jnp.float32),
          pltpu.VMEM((CHUNK, D), jnp.float32),
          pltpu.VMEM((e_per_w,), jnp.float32),
          pltpu.SemaphoreType.DMA,
      ],
  )
  def k(x_hbm, ei_hbm, et_hbm, w_hbm, out_hbm,
        sidx, didx, et_v, w_v, srows, drows, out_v, sem):
    wid = lax.axis_index("s") * NC + lax.axis_index("c")
    base = wid * e_per_w

    pltpu.sync_copy(ei_hbm.at[pl.ds(base, e_per_w)], sidx)
    pltpu.sync_copy(ei_hbm.at[pl.ds(E + base, e_per_w)], didx)
    pltpu.sync_copy(et_hbm.at[pl.ds(base, e_per_w)], et_v)
    pltpu.sync_copy(w_hbm, w_v)

    lane = lax.broadcasted_iota(jnp.int32, (L,), 0)

    def chunk_body(ch, carry):
      cbase = ch * CHUNK
      cp1 = pltpu.async_copy(x_hbm.at[sidx.at[pl.ds(cbase, CHUNK)]], srows, sem)
      cp2 = pltpu.async_copy(x_hbm.at[didx.at[pl.ds(cbase, CHUNK)]], drows, sem)
      cp1.wait()
      cp2.wait()

      for g in range(n_groups):
        eoff = cbase + g * L
        et16 = et_v[pl.ds(eoff, L)]
        erow = lane + g * L

        def d_body(d, acc):
          dv = jnp.full((L,), d, dtype=jnp.int32)
          s = plsc.load_gather(srows, [erow, dv])
          t = plsc.load_gather(drows, [erow, dv])
          w = plsc.load_gather(w_v, [et16, dv])
          return acc + s * t * w

        acc = lax.fori_loop(0, D, d_body, jnp.zeros((L,), jnp.float32))
        out_v[pl.ds(eoff, L)] = 1.0 / (1.0 + jnp.exp(-acc))
      return carry

    lax.fori_loop(0, n_chunks, chunk_body, 0)
    pltpu.sync_copy(out_v, out_hbm.at[pl.ds(base, e_per_w)])

  return k


def kernel(x, edge_index, edge_type, weight):
  E = edge_type.shape[0]
  D = x.shape[1]
  R = weight.shape[0]
  k = _sc_kernel(E, D, R, CHUNK=80)
  return k(x, edge_index.reshape(-1), edge_type, weight)


# speedup vs baseline: 1.2449x; 1.1328x over previous
import functools

import jax
import jax.numpy as jnp
from jax import lax
from jax.experimental import pallas as pl
from jax.experimental.pallas import tpu as pltpu
from jax.experimental.pallas import tpu_sc as plsc


def _sc_kernel(E, D, R, CHUNK):
  info = plsc.get_sparse_core_info()
  NC, NS, L = info.num_cores, info.num_subcores, info.num_lanes
  NW = NC * NS
  assert E % NW == 0
  e_per_w = E // NW
  assert e_per_w % CHUNK == 0 and CHUNK % L == 0
  n_chunks = e_per_w // CHUNK
  n_groups = CHUNK // L

  mesh = plsc.VectorSubcoreMesh(core_axis_name="c", subcore_axis_name="s")

  @functools.partial(
      pl.kernel,
      mesh=mesh,
      compiler_params=pltpu.CompilerParams(needs_layout_passes=False),
      out_type=jax.ShapeDtypeStruct((E,), jnp.float32),
      scratch_types=[
          pltpu.VMEM((e_per_w,), jnp.int32),
          pltpu.VMEM((e_per_w,), jnp.int32),
          pltpu.VMEM((e_per_w,), jnp.int32),
          pltpu.VMEM((R, D), jnp.float32),
          pltpu.VMEM((CHUNK, D), jnp.float32),
          pltpu.VMEM((CHUNK, D), jnp.float32),
          pltpu.VMEM((e_per_w,), jnp.float32),
          pltpu.SemaphoreType.DMA,
      ],
  )
  def k(x_hbm, ei_hbm, et_hbm, w_hbm, out_hbm,
        sidx, didx, et_v, w_v, srows, drows, out_v, sem):
    wid = lax.axis_index("s") * NC + lax.axis_index("c")
    base = wid * e_per_w

    pltpu.sync_copy(ei_hbm.at[pl.ds(base, e_per_w)], sidx)
    pltpu.sync_copy(ei_hbm.at[pl.ds(E + base, e_per_w)], didx)
    pltpu.sync_copy(et_hbm.at[pl.ds(base, e_per_w)], et_v)
    pltpu.sync_copy(w_hbm, w_v)

    lane = lax.broadcasted_iota(jnp.int32, (L,), 0)

    def chunk_body(ch, carry):
      cbase = ch * CHUNK
      cp1 = pltpu.async_copy(x_hbm.at[sidx.at[pl.ds(cbase, CHUNK)]], srows, sem)
      cp2 = pltpu.async_copy(x_hbm.at[didx.at[pl.ds(cbase, CHUNK)]], drows, sem)
      cp1.wait()
      cp2.wait()

      def group_body(g, _):
        eoff = cbase + g * L
        et16 = et_v[pl.ds(eoff, L)]
        erow = lane + g * L
        acc = jnp.zeros((L,), jnp.float32)
        for d in range(D):
          dv = jnp.full((L,), d, dtype=jnp.int32)
          s = plsc.load_gather(srows, [erow, dv])
          t = plsc.load_gather(drows, [erow, dv])
          w = plsc.load_gather(w_v, [et16, dv])
          acc = acc + s * t * w
        out_v[pl.ds(eoff, L)] = 1.0 / (1.0 + jnp.exp(-acc))
        return 0

      lax.fori_loop(0, n_groups, group_body, 0)
      return carry

    lax.fori_loop(0, n_chunks, chunk_body, 0)
    pltpu.sync_copy(out_v, out_hbm.at[pl.ds(base, e_per_w)])

  return k


def kernel(x, edge_index, edge_type, weight):
  E = edge_type.shape[0]
  D = x.shape[1]
  R = weight.shape[0]
  k = _sc_kernel(E, D, R, CHUNK=80)
  return k(x, edge_index.reshape(-1), edge_type, weight)


# speedup vs baseline: 6.0018x; 4.8212x over previous
import functools

import jax
import jax.numpy as jnp
from jax import lax
from jax.experimental import pallas as pl
from jax.experimental.pallas import tpu as pltpu
from jax.experimental.pallas import tpu_sc as plsc


def _sc_kernel(E, D, R, CHUNK):
  info = plsc.get_sparse_core_info()
  NC, NS, L = info.num_cores, info.num_subcores, info.num_lanes
  NW = NC * NS
  assert E % NW == 0
  e_per_w = E // NW
  assert e_per_w % CHUNK == 0 and CHUNK % L == 0
  n_chunks = e_per_w // CHUNK
  n_groups = CHUNK // L

  mesh = plsc.VectorSubcoreMesh(core_axis_name="c", subcore_axis_name="s")

  @functools.partial(
      pl.kernel,
      mesh=mesh,
      compiler_params=pltpu.CompilerParams(needs_layout_passes=False),
      out_type=jax.ShapeDtypeStruct((E,), jnp.float32),
      scratch_types=[
          pltpu.VMEM((e_per_w,), jnp.int32),
          pltpu.VMEM((e_per_w,), jnp.int32),
          pltpu.VMEM((e_per_w,), jnp.int32),
          pltpu.VMEM((R, D), jnp.float32),
          pltpu.VMEM((CHUNK, D), jnp.float32),
          pltpu.VMEM((CHUNK, D), jnp.float32),
          pltpu.VMEM((e_per_w,), jnp.float32),
          pltpu.SemaphoreType.DMA,
      ],
  )
  def k(x_hbm, ei_hbm, et_hbm, w_hbm, out_hbm,
        sidx, didx, et_v, w_v, srows, drows, out_v, sem):
    wid = lax.axis_index("s") * NC + lax.axis_index("c")
    base = wid * e_per_w

    pltpu.sync_copy(ei_hbm.at[pl.ds(base, e_per_w)], sidx)
    pltpu.sync_copy(ei_hbm.at[pl.ds(E + base, e_per_w)], didx)
    pltpu.sync_copy(et_hbm.at[pl.ds(base, e_per_w)], et_v)
    pltpu.sync_copy(w_hbm, w_v)

    lane = lax.broadcasted_iota(jnp.int32, (L,), 0)

    def chunk_body(ch, carry):
      cbase = ch * CHUNK
      cp1 = pltpu.async_copy(x_hbm.at[sidx.at[pl.ds(cbase, CHUNK)]], srows, sem)
      cp2 = pltpu.async_copy(x_hbm.at[didx.at[pl.ds(cbase, CHUNK)]], drows, sem)
      cp1.wait()
      cp2.wait()

      def group_body(g, _):
        eoff = cbase + g * L
        et16 = et_v[pl.ds(eoff, L)]
        erow = lane + g * L
        acc = jnp.zeros((L,), jnp.float32)
        for d in range(D):
          dv = (lane + d) & (D - 1)
          s = plsc.load_gather(srows, [erow, dv])
          t = plsc.load_gather(drows, [erow, dv])
          w = plsc.load_gather(w_v, [et16, dv])
          acc = acc + s * t * w
        out_v[pl.ds(eoff, L)] = 1.0 / (1.0 + jnp.exp(-acc))
        return 0

      lax.fori_loop(0, n_groups, group_body, 0)
      return carry

    lax.fori_loop(0, n_chunks, chunk_body, 0)
    pltpu.sync_copy(out_v, out_hbm.at[pl.ds(base, e_per_w)])

  return k


def kernel(x, edge_index, edge_type, weight):
  E = edge_type.shape[0]
  D = x.shape[1]
  R = weight.shape[0]
  k = _sc_kernel(E, D, R, CHUNK=80)
  return k(x, edge_index.reshape(-1), edge_type, weight)


# speedup vs baseline: 10.0272x; 1.6707x over previous
import functools

import jax
import jax.numpy as jnp
from jax import lax
from jax.experimental import pallas as pl
from jax.experimental.pallas import tpu as pltpu
from jax.experimental.pallas import tpu_sc as plsc


def _sc_kernel(E, D, R, CHUNK):
  info = plsc.get_sparse_core_info()
  NC, NS, L = info.num_cores, info.num_subcores, info.num_lanes
  NW = NC * NS
  assert E % NW == 0
  e_per_w = E // NW
  assert e_per_w % CHUNK == 0 and CHUNK % L == 0
  n_chunks = e_per_w // CHUNK
  n_groups = CHUNK // L

  mesh = plsc.VectorSubcoreMesh(core_axis_name="c", subcore_axis_name="s")

  @functools.partial(
      pl.kernel,
      mesh=mesh,
      compiler_params=pltpu.CompilerParams(needs_layout_passes=False),
      out_type=jax.ShapeDtypeStruct((E,), jnp.float32),
      scratch_types=[
          pltpu.VMEM((e_per_w,), jnp.int32),
          pltpu.VMEM((e_per_w,), jnp.int32),
          pltpu.VMEM((e_per_w,), jnp.int32),
          pltpu.VMEM((R, D), jnp.float32),
          pltpu.VMEM((CHUNK, D), jnp.float32),
          pltpu.VMEM((CHUNK, D), jnp.float32),
          pltpu.VMEM((CHUNK, D), jnp.float32),
          pltpu.VMEM((CHUNK, D), jnp.float32),
          pltpu.VMEM((e_per_w,), jnp.float32),
          pltpu.SemaphoreType.DMA,
          pltpu.SemaphoreType.DMA,
          pltpu.SemaphoreType.DMA,
          pltpu.SemaphoreType.DMA,
      ],
  )
  def k(x_hbm, ei_hbm, et_hbm, w_hbm, out_hbm,
        sidx, didx, et_v, w_v, srows0, drows0, srows1, drows1, out_v,
        ss0, sd0, ss1, sd1):
    wid = lax.axis_index("s") * NC + lax.axis_index("c")
    base = wid * e_per_w

    pltpu.sync_copy(ei_hbm.at[pl.ds(base, e_per_w)], sidx)
    pltpu.sync_copy(ei_hbm.at[pl.ds(E + base, e_per_w)], didx)
    pltpu.sync_copy(et_hbm.at[pl.ds(base, e_per_w)], et_v)
    pltpu.sync_copy(w_hbm, w_v)

    lane = lax.broadcasted_iota(jnp.int32, (L,), 0)
    bufs = ((srows0, drows0, ss0, sd0), (srows1, drows1, ss1, sd1))

    def gather_pair(ch, b):
      sr, dr, ss, sd = bufs[b]
      cbase = ch * CHUNK
      pltpu.async_copy(x_hbm.at[sidx.at[pl.ds(cbase, CHUNK)]], sr, ss)
      pltpu.async_copy(x_hbm.at[didx.at[pl.ds(cbase, CHUNK)]], dr, sd)

    def wait_pair(ch, b):
      sr, dr, ss, sd = bufs[b]
      cbase = ch * CHUNK
      pltpu.make_async_copy(x_hbm.at[sidx.at[pl.ds(cbase, CHUNK)]], sr, ss).wait()
      pltpu.make_async_copy(x_hbm.at[didx.at[pl.ds(cbase, CHUNK)]], dr, sd).wait()

    def compute(ch, b):
      sr, dr, _, _ = bufs[b]
      cbase = ch * CHUNK

      def group_body(g, _):
        eoff = cbase + g * L
        et16 = et_v[pl.ds(eoff, L)]
        erow = lane + g * L
        acc = jnp.zeros((L,), jnp.float32)
        for d in range(D):
          dv = (lane + d) & (D - 1)
          s = plsc.load_gather(sr, [erow, dv])
          t = plsc.load_gather(dr, [erow, dv])
          w = plsc.load_gather(w_v, [et16, dv])
          acc = acc + s * t * w
        out_v[pl.ds(eoff, L)] = 1.0 / (1.0 + jnp.exp(-acc))
        return 0

      lax.fori_loop(0, n_groups, group_body, 0)

    assert n_chunks % 2 == 1
    gather_pair(0, 0)

    def pair_body(chp, carry):
      ch0 = 2 * chp
      wait_pair(ch0, 0)
      gather_pair(ch0 + 1, 1)
      compute(ch0, 0)
      wait_pair(ch0 + 1, 1)
      gather_pair(ch0 + 2, 0)
      compute(ch0 + 1, 1)
      return carry

    lax.fori_loop(0, (n_chunks - 1) // 2, pair_body, 0)
    wait_pair(n_chunks - 1, 0)
    compute(n_chunks - 1, 0)
    pltpu.sync_copy(out_v, out_hbm.at[pl.ds(base, e_per_w)])

  return k


def kernel(x, edge_index, edge_type, weight):
  E = edge_type.shape[0]
  D = x.shape[1]
  R = weight.shape[0]
  k = _sc_kernel(E, D, R, CHUNK=80)
  return k(x, edge_index.reshape(-1), edge_type, weight)
